# baseline (speedup 1.0000x reference)
import functools

import jax
import jax.numpy as jnp
from jax import lax
from jax.experimental import pallas as pl
from jax.experimental.pallas import tpu as pltpu
from jax.experimental.pallas import tpu_sc as plsc

N_BASE = 5000
N_JOINT = 20000
N_FOOT = 5000
H = 128
F32 = jnp.float32

E_B2J, E_J2J, E_J2F = 120000, 200000, 80000
PAD_B2J = 131072
PAD_J2J = 212992
PAD_J2F = 98304
SENTINEL = 1 << 28

ACC_ROWS = 10112
ACCF_ROWS = 5120
JHALF = 10000
TRASH_J = 10000
TRASH_F = 5000


def _cdiv(a, b):
    return (a + b - 1) // b


def _tc_joint_body(x_ref, we_ref, be_ref, w1_ref, w2_ref, wr_ref, br_ref,
                   m1_ref, m2_ref, r_ref):
    x = jnp.maximum(
        jnp.dot(x_ref[:], we_ref[:], preferred_element_type=F32) + be_ref[:], 0.0)
    m1_ref[:] = jnp.dot(x, w1_ref[:], preferred_element_type=F32)
    m2_ref[:] = jnp.dot(x, w2_ref[:], preferred_element_type=F32)
    r_ref[:] = jnp.dot(x, wr_ref[:], preferred_element_type=F32) + br_ref[:]


def _tc_src_body(x_ref, we_ref, be_ref, w1_ref, m1_ref):
    x = jnp.maximum(
        jnp.dot(x_ref[:], we_ref[:], preferred_element_type=F32) + be_ref[:], 0.0)
    m1_ref[:] = jnp.dot(x, w1_ref[:], preferred_element_type=F32)


def _tc_foot_body(x_ref, we_ref, be_ref, w1_ref, wr_ref, br_ref, m1_ref, r_ref):
    x = jnp.maximum(
        jnp.dot(x_ref[:], we_ref[:], preferred_element_type=F32) + be_ref[:], 0.0)
    m1_ref[:] = jnp.dot(x, w1_ref[:], preferred_element_type=F32)
    r_ref[:] = jnp.dot(x, wr_ref[:], preferred_element_type=F32) + br_ref[:]


def _tc_update_body(a_ref, r_ref, w_ref, o_ref):
    x = jnp.maximum(a_ref[:] + r_ref[:], 0.0)
    o_ref[:] = jnp.dot(x, w_ref[:], preferred_element_type=F32)


def _tc_update2_body(a0_ref, a1_ref, r_ref, w_ref, b_ref, o_ref):
    x = jnp.maximum(a0_ref[:] + a1_ref[:] + r_ref[:], 0.0)
    o_ref[:] = jnp.dot(x, w_ref[:], preferred_element_type=F32) + b_ref[:]


def _tc_final_body(a0_ref, a1_ref, r_ref, w_ref, b_ref, o_ref):
    x = jnp.maximum(a0_ref[:] + a1_ref[:] + r_ref[:], 0.0)
    o_ref[:] = jnp.dot(x, w_ref[:], preferred_element_type=F32) + b_ref[:]


def _row_spec(blk):
    return pl.BlockSpec((blk, H), lambda i: (i, 0))


def _w_spec():
    return pl.BlockSpec((H, H), lambda i: (0, 0))


def _b_spec():
    return pl.BlockSpec((1, H), lambda i: (0, 0))


def _tc_call(body, n_rows, blk, n_out, in_specs, out_last=None):
    out_specs = [_row_spec(blk)] * n_out
    out_shape = [jax.ShapeDtypeStruct((n_rows, H), F32)] * n_out
    if out_last is not None:
        out_specs[-1] = pl.BlockSpec((blk, out_last), lambda i: (i, 0))
        out_shape[-1] = jax.ShapeDtypeStruct((n_rows, out_last), F32)
    return pl.pallas_call(
        body,
        grid=(n_rows // blk,),
        in_specs=in_specs,
        out_specs=out_specs,
        out_shape=out_shape,
    )


def _zero_rows(rows_ref, n_rows):
    z = jnp.zeros((16,), F32)

    def body(r, c):
        for cc in range(8):
            rows_ref[r, pl.ds(cc * 16, 16)] = z
        return c

    lax.fori_loop(0, n_rows, body, 0)


def _zero_spmem(rows_ref, acc_ref, base, total):
    off = 0
    while off < total:
        n = min(256, total - off)
        pltpu.sync_copy(rows_ref.at[pl.ds(0, n)], acc_ref.at[pl.ds(base + off, n)])
        off += n


def _process_edges(src_ref, dst_ref, table_ref, acc_ref, lo, hi, trash,
                   n_sc, chunk_base, idx_s_ref, idx_d_ref, idx_l_ref,
                   rows_ref, sem):
    def one(j, c):
        row0 = (chunk_base + j) * 8
        pltpu.sync_copy(src_ref.at[pl.ds(row0, 8)], idx_s_ref)
        pltpu.sync_copy(dst_ref.at[pl.ds(row0, 8)], idx_d_ref)
        for rnd in range(4):
            cps = [
                pltpu.async_copy(table_ref.at[idx_s_ref.at[rnd * 2 + k]],
                                 rows_ref.at[pl.ds(k * 128, 128)], sem)
                for k in range(2)
            ]
            for k in range(2):
                g = rnd * 2 + k
                for cc in range(8):
                    d = idx_d_ref[g, pl.ds(cc * 16, 16)]
                    ok = (d >= lo) & (d < hi)
                    dl = jnp.where(ok, d - lo, trash + (d & 63))
                    idx_l_ref[g, pl.ds(cc * 16, 16)] = dl
            for k in range(2):
                cps[k].wait()
                pltpu.sync_copy(rows_ref.at[pl.ds(k * 128, 128)],
                                acc_ref.at[idx_l_ref.at[rnd * 2 + k]],
                                add=True)
        return c

    lax.fori_loop(0, n_sc, one, 0)


def _sc_layer0(m_b2j, m_j2j, m_f2j, m_j2f,
               sb_s, sb_d, sj_s, sj_d, sf_s, sf_d, sjf_s, sjf_d):
    mesh = plsc.VectorSubcoreMesh(core_axis_name="c", subcore_axis_name="s")

    @functools.partial(
        pl.kernel,
        mesh=mesh,
        out_type=[
            jax.ShapeDtypeStruct((N_JOINT, H), F32),
            jax.ShapeDtypeStruct((N_FOOT, H), F32),
            jax.ShapeDtypeStruct((N_FOOT, H), F32),
        ],
        scratch_types=[
            pltpu.VMEM_SHARED((ACC_ROWS, H), F32),
            pltpu.VMEM((8, 128), jnp.int32),
            pltpu.VMEM((8, 128), jnp.int32),
            pltpu.VMEM((8, 128), jnp.int32),
            pltpu.VMEM((256, H), F32),
            pltpu.SemaphoreType.DMA,
        ],
    )
    def k(mb_ref, mj_ref, mf_ref, mjf_ref,
          sbs_ref, sbd_ref, sjs_ref, sjd_ref, sfs_ref, sfd_ref,
          sjfs_ref, sjfd_ref,
          accj_out, accfa_out, accfb_out,
          acc_sh, idx_s, idx_d, idx_l, rows, sem):
        cid = lax.axis_index("c")
        sid = lax.axis_index("s")
        wid = cid * 16 + sid

        _zero_rows(rows, 256)
        _zero_spmem(rows, acc_sh, sid * 632, 632)
        plsc.subcore_barrier()

        lo = cid * JHALF
        hi = lo + JHALF
        _process_edges(sbs_ref, sbd_ref, mb_ref, acc_sh, lo, hi, TRASH_J,
                       8, sid * 8, idx_s, idx_d, idx_l, rows, sem)
        _process_edges(sjs_ref, sjd_ref, mj_ref, acc_sh, lo, hi, TRASH_J,
                       13, sid * 13, idx_s, idx_d, idx_l, rows, sem)
        _process_edges(sfs_ref, sfd_ref, mf_ref, acc_sh, lo, hi, TRASH_J,
                       6, sid * 6, idx_s, idx_d, idx_l, rows, sem)
        plsc.subcore_barrier()

        pltpu.sync_copy(acc_sh.at[pl.ds(sid * 624, 624)],
                        accj_out.at[pl.ds(cid * JHALF + sid * 624, 624)])

        @pl.when(sid == 0)
        def _():
            pltpu.sync_copy(acc_sh.at[pl.ds(9984, 16)],
                            accj_out.at[pl.ds(cid * JHALF + 9984, 16)])

        plsc.subcore_barrier()
        _zero_rows(rows, 256)
        _zero_spmem(rows, acc_sh, sid * 320, 320)
        plsc.subcore_barrier()

        _process_edges(sjfs_ref, sjfd_ref, mjf_ref, acc_sh, 0, N_FOOT,
                       TRASH_F, 3, wid * 3, idx_s, idx_d, idx_l, rows, sem)
        plsc.subcore_barrier()

        @pl.when(cid == 0)
        def _():
            pltpu.sync_copy(acc_sh.at[pl.ds(sid * 312, 312)],
                            accfa_out.at[pl.ds(sid * 312, 312)])

            @pl.when(sid == 0)
            def _():
                pltpu.sync_copy(acc_sh.at[pl.ds(4992, 8)],
                                accfa_out.at[pl.ds(4992, 8)])

        @pl.when(cid == 1)
        def _():
            pltpu.sync_copy(acc_sh.at[pl.ds(sid * 312, 312)],
                            accfb_out.at[pl.ds(sid * 312, 312)])

            @pl.when(sid == 0)
            def _():
                pltpu.sync_copy(acc_sh.at[pl.ds(4992, 8)],
                                accfb_out.at[pl.ds(4992, 8)])

    return k(m_b2j, m_j2j, m_f2j, m_j2f,
             sb_s, sb_d, sj_s, sj_d, sf_s, sf_d, sjf_s, sjf_d)


def _sc_layer1(m2, sjf_s, sjf_d):
    mesh = plsc.VectorSubcoreMesh(core_axis_name="c", subcore_axis_name="s")

    @functools.partial(
        pl.kernel,
        mesh=mesh,
        out_type=[
            jax.ShapeDtypeStruct((N_FOOT, H), F32),
            jax.ShapeDtypeStruct((N_FOOT, H), F32),
        ],
        scratch_types=[
            pltpu.VMEM_SHARED((ACCF_ROWS, H), F32),
            pltpu.VMEM((8, 128), jnp.int32),
            pltpu.VMEM((8, 128), jnp.int32),
            pltpu.VMEM((8, 128), jnp.int32),
            pltpu.VMEM((256, H), F32),
            pltpu.SemaphoreType.DMA,
        ],
    )
    def k(m_ref, ss_ref, sd_ref, acca_out, accb_out,
          accf_sh, idx_s, idx_d, idx_l, rows, sem):
        cid = lax.axis_index("c")
        sid = lax.axis_index("s")
        wid = cid * 16 + sid

        _zero_rows(rows, 256)
        _zero_spmem(rows, accf_sh, sid * 320, 320)
        plsc.subcore_barrier()

        _process_edges(ss_ref, sd_ref, m_ref, accf_sh, 0, N_FOOT, TRASH_F,
                       3, wid * 3, idx_s, idx_d, idx_l, rows, sem)
        plsc.subcore_barrier()

        @pl.when(cid == 0)
        def _():
            pltpu.sync_copy(accf_sh.at[pl.ds(sid * 312, 312)],
                            acca_out.at[pl.ds(sid * 312, 312)])

            @pl.when(sid == 0)
            def _():
                pltpu.sync_copy(accf_sh.at[pl.ds(4992, 8)],
                                acca_out.at[pl.ds(4992, 8)])

        @pl.when(cid == 1)
        def _():
            pltpu.sync_copy(accf_sh.at[pl.ds(sid * 312, 312)],
                            accb_out.at[pl.ds(sid * 312, 312)])

            @pl.when(sid == 0)
            def _():
                pltpu.sync_copy(accf_sh.at[pl.ds(4992, 8)],
                                accb_out.at[pl.ds(4992, 8)])

    return k(m2, sjf_s, sjf_d)


def _prep_edges(ei, pad_to):
    e = ei.shape[1]
    src = jnp.concatenate(
        [ei[0].astype(jnp.int32), jnp.zeros((pad_to - e,), jnp.int32)])
    dst = jnp.concatenate(
        [ei[1].astype(jnp.int32),
         jnp.full((pad_to - e,), SENTINEL, jnp.int32)])
    return src.reshape(pad_to // 128, 128), dst.reshape(pad_to // 128, 128)


def kernel(x_base, x_joint, x_foot, ei_b2j, ei_j2b, ei_j2j, ei_j2f, ei_f2j,
           W_enc, b_enc, W_rel, b_rel, W_root, W_dec, b_dec):
    del ei_j2b

    wroot_j = W_root[0, 0] + W_root[0, 2] + W_root[0, 4]
    bias_j = (b_rel[0, 0] + b_rel[0, 2] + b_rel[0, 4]).reshape(1, H)

    sb_s, sb_d = _prep_edges(ei_b2j, PAD_B2J)
    sj_s, sj_d = _prep_edges(ei_j2j, PAD_J2J)
    sf_s, sf_d = _prep_edges(ei_f2j, PAD_J2F)
    sjf_s, sjf_d = _prep_edges(ei_j2f, PAD_J2F)

    (m_b2j,) = _tc_call(
        _tc_src_body, N_BASE, 1000, 1,
        [_row_spec(1000), _w_spec(), _b_spec(), _w_spec()],
    )(x_base, W_enc[0], b_enc[0].reshape(1, H), W_rel[0, 0])

    m_j2j, m_j2f, r_j = _tc_call(
        _tc_joint_body, N_JOINT, 1000, 3,
        [_row_spec(1000), _w_spec(), _b_spec(), _w_spec(), _w_spec(),
         _w_spec(), _b_spec()],
    )(x_joint, W_enc[1], b_enc[1].reshape(1, H), W_rel[0, 2], W_rel[0, 3],
      wroot_j, bias_j)

    m_f2j, r_f = _tc_call(
        _tc_foot_body, N_FOOT, 1000, 2,
        [_row_spec(1000), _w_spec(), _b_spec(), _w_spec(), _w_spec(),
         _b_spec()],
    )(x_foot, W_enc[2], b_enc[2].reshape(1, H), W_rel[0, 4], W_root[0, 3],
      b_rel[0, 3].reshape(1, H))

    acc_j, acc_fa, acc_fb = _sc_layer0(
        m_b2j, m_j2j, m_f2j, m_j2f,
        sb_s, sb_d, sj_s, sj_d, sf_s, sf_d, sjf_s, sjf_d)

    (m2,) = _tc_call(
        _tc_update_body, N_JOINT, 1000, 1,
        [_row_spec(1000), _row_spec(1000), _w_spec()],
    )(acc_j, r_j, W_rel[1, 3])

    (r2_f,) = _tc_call(
        _tc_update2_body, N_FOOT, 1000, 1,
        [_row_spec(1000), _row_spec(1000), _row_spec(1000), _w_spec(),
         _b_spec()],
    )(acc_fa, acc_fb, r_f, W_root[1, 3], b_rel[1, 3].reshape(1, H))

    acc2_a, acc2_b = _sc_layer1(m2, sjf_s, sjf_d)

    (y,) = _tc_call(
        _tc_final_body, N_FOOT, 1000, 1,
        [_row_spec(1000), _row_spec(1000), _row_spec(1000),
         pl.BlockSpec((H, 1), lambda i: (0, 0)),
         pl.BlockSpec((1, 1), lambda i: (0, 0))],
        out_last=1,
    )(acc2_a, acc2_b, r2_f, W_dec, b_dec.reshape(1, 1))
    return y


# speedup vs baseline: 1.0002x; 1.0002x over previous
import functools

import jax
import jax.numpy as jnp
from jax import lax
from jax.experimental import pallas as pl
from jax.experimental.pallas import tpu as pltpu
from jax.experimental.pallas import tpu_sc as plsc

N_BASE = 5000
N_JOINT = 20000
N_FOOT = 5000
H = 128
F32 = jnp.float32

E_B2J, E_J2J, E_J2F = 120000, 200000, 80000
PAD_B2J = 131072
PAD_J2J = 212992
PAD_J2F = 98304
SENTINEL = 1 << 28

ACC_ROWS = 10112
ACCF_ROWS = 5120
JHALF = 10000
TRASH_J = 10000
TRASH_F = 5000


def _cdiv(a, b):
    return (a + b - 1) // b


def _tc_joint_body(x_ref, we_ref, be_ref, w1_ref, w2_ref, wr_ref, br_ref,
                   m1_ref, m2_ref, r_ref):
    x = jnp.maximum(
        jnp.dot(x_ref[:], we_ref[:], preferred_element_type=F32) + be_ref[:], 0.0)
    m1_ref[:] = jnp.dot(x, w1_ref[:], preferred_element_type=F32)
    m2_ref[:] = jnp.dot(x, w2_ref[:], preferred_element_type=F32)
    r_ref[:] = jnp.dot(x, wr_ref[:], preferred_element_type=F32) + br_ref[:]


def _tc_src_body(x_ref, we_ref, be_ref, w1_ref, m1_ref):
    x = jnp.maximum(
        jnp.dot(x_ref[:], we_ref[:], preferred_element_type=F32) + be_ref[:], 0.0)
    m1_ref[:] = jnp.dot(x, w1_ref[:], preferred_element_type=F32)


def _tc_foot_body(x_ref, we_ref, be_ref, w1_ref, wr_ref, br_ref, m1_ref, r_ref):
    x = jnp.maximum(
        jnp.dot(x_ref[:], we_ref[:], preferred_element_type=F32) + be_ref[:], 0.0)
    m1_ref[:] = jnp.dot(x, w1_ref[:], preferred_element_type=F32)
    r_ref[:] = jnp.dot(x, wr_ref[:], preferred_element_type=F32) + br_ref[:]


def _tc_update_body(a_ref, r_ref, w_ref, o_ref):
    x = jnp.maximum(a_ref[:] + r_ref[:], 0.0)
    o_ref[:] = jnp.dot(x, w_ref[:], preferred_element_type=F32)


def _tc_update2_body(a0_ref, a1_ref, r_ref, w_ref, b_ref, o_ref):
    x = jnp.maximum(a0_ref[:] + a1_ref[:] + r_ref[:], 0.0)
    o_ref[:] = jnp.dot(x, w_ref[:], preferred_element_type=F32) + b_ref[:]


def _tc_final_body(a0_ref, a1_ref, r_ref, w_ref, b_ref, o_ref):
    x = jnp.maximum(a0_ref[:] + a1_ref[:] + r_ref[:], 0.0)
    o_ref[:] = jnp.dot(x, w_ref[:], preferred_element_type=F32) + b_ref[:]


def _row_spec(blk):
    return pl.BlockSpec((blk, H), lambda i: (i, 0))


def _w_spec():
    return pl.BlockSpec((H, H), lambda i: (0, 0))


def _b_spec():
    return pl.BlockSpec((1, H), lambda i: (0, 0))


def _tc_call(body, n_rows, blk, n_out, in_specs, out_last=None):
    out_specs = [_row_spec(blk)] * n_out
    out_shape = [jax.ShapeDtypeStruct((n_rows, H), F32)] * n_out
    if out_last is not None:
        out_specs[-1] = pl.BlockSpec((blk, out_last), lambda i: (i, 0))
        out_shape[-1] = jax.ShapeDtypeStruct((n_rows, out_last), F32)
    return pl.pallas_call(
        body,
        grid=(n_rows // blk,),
        in_specs=in_specs,
        out_specs=out_specs,
        out_shape=out_shape,
    )


def _zero_rows(rows_ref, n_rows):
    z = jnp.zeros((16,), F32)

    def body(r, c):
        for cc in range(8):
            rows_ref[r, pl.ds(cc * 16, 16)] = z
        return c

    lax.fori_loop(0, n_rows, body, 0)


def _zero_spmem(rows_ref, acc_ref, base, total):
    off = 0
    while off < total:
        n = min(256, total - off)
        pltpu.sync_copy(rows_ref.at[pl.ds(0, n)], acc_ref.at[pl.ds(base + off, n)])
        off += n


def _process_edges(pk_ref, table_ref, acc_ref, lo, hi, trash,
                   n_sc, chunk_base, idx_ref, idx_l_ref,
                   rows_ref, sem_g, sem_s):
    def one(j, c):
        row0 = (chunk_base + j) * 16
        pltpu.sync_copy(pk_ref.at[pl.ds(row0, 16)], idx_ref)
        for g in range(8):
            for cc in range(8):
                d = idx_ref[8 + g, pl.ds(cc * 16, 16)]
                ok = (d >= lo) & (d < hi)
                dl = jnp.where(ok, d - lo, trash + (d & 63))
                idx_l_ref[g, pl.ds(cc * 16, 16)] = dl

        def gather(k):
            return pltpu.async_copy(table_ref.at[idx_ref.at[k]],
                                    rows_ref.at[pl.ds((k % 2) * 128, 128)],
                                    sem_g)

        gs = {0: gather(0), 1: gather(1)}
        ss = {}
        for k in range(8):
            gs[k].wait()
            ss[k] = pltpu.async_copy(rows_ref.at[pl.ds((k % 2) * 128, 128)],
                                     acc_ref.at[idx_l_ref.at[k]],
                                     sem_s, add=True)
            if k >= 1 and k + 1 < 8:
                ss[k - 1].wait()
                gs[k + 1] = gather(k + 1)
        ss[6].wait()
        ss[7].wait()
        return c

    lax.fori_loop(0, n_sc, one, 0)


def _sc_layer0(m_b2j, m_j2j, m_f2j, m_j2f, pk_b, pk_j, pk_f, pk_jf):
    mesh = plsc.VectorSubcoreMesh(core_axis_name="c", subcore_axis_name="s")

    @functools.partial(
        pl.kernel,
        mesh=mesh,
        out_type=[
            jax.ShapeDtypeStruct((N_JOINT, H), F32),
            jax.ShapeDtypeStruct((N_FOOT, H), F32),
            jax.ShapeDtypeStruct((N_FOOT, H), F32),
        ],
        scratch_types=[
            pltpu.VMEM_SHARED((ACC_ROWS, H), F32),
            pltpu.VMEM((16, 128), jnp.int32),
            pltpu.VMEM((8, 128), jnp.int32),
            pltpu.VMEM((256, H), F32),
            pltpu.SemaphoreType.DMA,
            pltpu.SemaphoreType.DMA,
        ],
    )
    def k(mb_ref, mj_ref, mf_ref, mjf_ref,
          pkb_ref, pkj_ref, pkf_ref, pkjf_ref,
          accj_out, accfa_out, accfb_out,
          acc_sh, idx, idx_l, rows, sem_g, sem_s):
        cid = lax.axis_index("c")
        sid = lax.axis_index("s")
        wid = cid * 16 + sid

        _zero_rows(rows, 256)
        _zero_spmem(rows, acc_sh, sid * 632, 632)
        plsc.subcore_barrier()

        lo = cid * JHALF
        hi = lo + JHALF
        _process_edges(pkb_ref, mb_ref, acc_sh, lo, hi, TRASH_J,
                       8, sid * 8, idx, idx_l, rows, sem_g, sem_s)
        _process_edges(pkj_ref, mj_ref, acc_sh, lo, hi, TRASH_J,
                       13, sid * 13, idx, idx_l, rows, sem_g, sem_s)
        _process_edges(pkf_ref, mf_ref, acc_sh, lo, hi, TRASH_J,
                       6, sid * 6, idx, idx_l, rows, sem_g, sem_s)
        plsc.subcore_barrier()

        pltpu.sync_copy(acc_sh.at[pl.ds(sid * 624, 624)],
                        accj_out.at[pl.ds(cid * JHALF + sid * 624, 624)])

        @pl.when(sid == 0)
        def _():
            pltpu.sync_copy(acc_sh.at[pl.ds(9984, 16)],
                            accj_out.at[pl.ds(cid * JHALF + 9984, 16)])

        plsc.subcore_barrier()
        _zero_rows(rows, 256)
        _zero_spmem(rows, acc_sh, sid * 320, 320)
        plsc.subcore_barrier()

        _process_edges(pkjf_ref, mjf_ref, acc_sh, 0, N_FOOT, TRASH_F,
                       3, wid * 3, idx, idx_l, rows, sem_g, sem_s)
        plsc.subcore_barrier()

        @pl.when(cid == 0)
        def _():
            pltpu.sync_copy(acc_sh.at[pl.ds(sid * 312, 312)],
                            accfa_out.at[pl.ds(sid * 312, 312)])

            @pl.when(sid == 0)
            def _():
                pltpu.sync_copy(acc_sh.at[pl.ds(4992, 8)],
                                accfa_out.at[pl.ds(4992, 8)])

        @pl.when(cid == 1)
        def _():
            pltpu.sync_copy(acc_sh.at[pl.ds(sid * 312, 312)],
                            accfb_out.at[pl.ds(sid * 312, 312)])

            @pl.when(sid == 0)
            def _():
                pltpu.sync_copy(acc_sh.at[pl.ds(4992, 8)],
                                accfb_out.at[pl.ds(4992, 8)])

    return k(m_b2j, m_j2j, m_f2j, m_j2f, pk_b, pk_j, pk_f, pk_jf)


def _sc_layer1(m2, pk_jf):
    mesh = plsc.VectorSubcoreMesh(core_axis_name="c", subcore_axis_name="s")

    @functools.partial(
        pl.kernel,
        mesh=mesh,
        out_type=[
            jax.ShapeDtypeStruct((N_FOOT, H), F32),
            jax.ShapeDtypeStruct((N_FOOT, H), F32),
        ],
        scratch_types=[
            pltpu.VMEM_SHARED((ACCF_ROWS, H), F32),
            pltpu.VMEM((16, 128), jnp.int32),
            pltpu.VMEM((8, 128), jnp.int32),
            pltpu.VMEM((256, H), F32),
            pltpu.SemaphoreType.DMA,
            pltpu.SemaphoreType.DMA,
        ],
    )
    def k(m_ref, pk_ref, acca_out, accb_out,
          accf_sh, idx, idx_l, rows, sem_g, sem_s):
        cid = lax.axis_index("c")
        sid = lax.axis_index("s")
        wid = cid * 16 + sid

        _zero_rows(rows, 256)
        _zero_spmem(rows, accf_sh, sid * 320, 320)
        plsc.subcore_barrier()

        _process_edges(pk_ref, m_ref, accf_sh, 0, N_FOOT, TRASH_F,
                       3, wid * 3, idx, idx_l, rows, sem_g, sem_s)
        plsc.subcore_barrier()

        @pl.when(cid == 0)
        def _():
            pltpu.sync_copy(accf_sh.at[pl.ds(sid * 312, 312)],
                            acca_out.at[pl.ds(sid * 312, 312)])

            @pl.when(sid == 0)
            def _():
                pltpu.sync_copy(accf_sh.at[pl.ds(4992, 8)],
                                acca_out.at[pl.ds(4992, 8)])

        @pl.when(cid == 1)
        def _():
            pltpu.sync_copy(accf_sh.at[pl.ds(sid * 312, 312)],
                            accb_out.at[pl.ds(sid * 312, 312)])

            @pl.when(sid == 0)
            def _():
                pltpu.sync_copy(accf_sh.at[pl.ds(4992, 8)],
                                accb_out.at[pl.ds(4992, 8)])

    return k(m2, pk_jf)


def _prep_edges(ei, pad_to):
    e = ei.shape[1]
    src = jnp.concatenate(
        [ei[0].astype(jnp.int32), jnp.zeros((pad_to - e,), jnp.int32)])
    dst = jnp.concatenate(
        [ei[1].astype(jnp.int32),
         jnp.full((pad_to - e,), SENTINEL, jnp.int32)])
    packed = jnp.concatenate(
        [src.reshape(-1, 8, 128), dst.reshape(-1, 8, 128)], axis=1)
    return packed.reshape(-1, 128)


def kernel(x_base, x_joint, x_foot, ei_b2j, ei_j2b, ei_j2j, ei_j2f, ei_f2j,
           W_enc, b_enc, W_rel, b_rel, W_root, W_dec, b_dec):
    del ei_j2b

    wroot_j = W_root[0, 0] + W_root[0, 2] + W_root[0, 4]
    bias_j = (b_rel[0, 0] + b_rel[0, 2] + b_rel[0, 4]).reshape(1, H)

    pk_b = _prep_edges(ei_b2j, PAD_B2J)
    pk_j = _prep_edges(ei_j2j, PAD_J2J)
    pk_f = _prep_edges(ei_f2j, PAD_J2F)
    pk_jf = _prep_edges(ei_j2f, PAD_J2F)

    (m_b2j,) = _tc_call(
        _tc_src_body, N_BASE, 1000, 1,
        [_row_spec(1000), _w_spec(), _b_spec(), _w_spec()],
    )(x_base, W_enc[0], b_enc[0].reshape(1, H), W_rel[0, 0])

    m_j2j, m_j2f, r_j = _tc_call(
        _tc_joint_body, N_JOINT, 1000, 3,
        [_row_spec(1000), _w_spec(), _b_spec(), _w_spec(), _w_spec(),
         _w_spec(), _b_spec()],
    )(x_joint, W_enc[1], b_enc[1].reshape(1, H), W_rel[0, 2], W_rel[0, 3],
      wroot_j, bias_j)

    m_f2j, r_f = _tc_call(
        _tc_foot_body, N_FOOT, 1000, 2,
        [_row_spec(1000), _w_spec(), _b_spec(), _w_spec(), _w_spec(),
         _b_spec()],
    )(x_foot, W_enc[2], b_enc[2].reshape(1, H), W_rel[0, 4], W_root[0, 3],
      b_rel[0, 3].reshape(1, H))

    acc_j, acc_fa, acc_fb = _sc_layer0(
        m_b2j, m_j2j, m_f2j, m_j2f, pk_b, pk_j, pk_f, pk_jf)

    (m2,) = _tc_call(
        _tc_update_body, N_JOINT, 1000, 1,
        [_row_spec(1000), _row_spec(1000), _w_spec()],
    )(acc_j, r_j, W_rel[1, 3])

    (r2_f,) = _tc_call(
        _tc_update2_body, N_FOOT, 1000, 1,
        [_row_spec(1000), _row_spec(1000), _row_spec(1000), _w_spec(),
         _b_spec()],
    )(acc_fa, acc_fb, r_f, W_root[1, 3], b_rel[1, 3].reshape(1, H))

    acc2_a, acc2_b = _sc_layer1(m2, pk_jf)

    (y,) = _tc_call(
        _tc_final_body, N_FOOT, 1000, 1,
        [_row_spec(1000), _row_spec(1000), _row_spec(1000),
         pl.BlockSpec((H, 1), lambda i: (0, 0)),
         pl.BlockSpec((1, 1), lambda i: (0, 0))],
        out_last=1,
    )(acc2_a, acc2_b, r2_f, W_dec, b_dec.reshape(1, 1))
    return y
